# baseline (speedup 1.0000x reference)
import functools

import jax
import jax.numpy as jnp
from jax import lax
from jax.experimental import pallas as pl
from jax.experimental.pallas import tpu as pltpu
from jax.experimental.pallas import tpu_sc as plsc

_B = 16384
_D = 64
_CHUNK = 128


def kernel(user_table, item_table, user_idx, item_idx):
    info = plsc.get_sparse_core_info()
    nw = info.num_cores * info.num_subcores
    bpw = _B // nw
    nchunk = bpw // _CHUNK

    mesh = plsc.VectorSubcoreMesh(core_axis_name="c", subcore_axis_name="s")

    @functools.partial(
        pl.kernel,
        mesh=mesh,
        out_type=jax.ShapeDtypeStruct((2, _B, _D), jnp.float32),
        scratch_types=[
            pltpu.VMEM((bpw,), jnp.int32),
            pltpu.VMEM((bpw,), jnp.int32),
            pltpu.VMEM((bpw, _D), jnp.float32),
            pltpu.VMEM((bpw, _D), jnp.float32),
            pltpu.SemaphoreType.DMA((2, 4)),
            pltpu.SemaphoreType.DMA,
        ],
        compiler_params=pltpu.CompilerParams(use_tc_tiling_on_sc=False),
    )
    def _emb(ut, it, ui, ii, out, uidx_v, iidx_v, urows_v, irows_v, gsem, wsem):
        wid = lax.axis_index("s") * info.num_cores + lax.axis_index("c")
        base = wid * bpw
        pltpu.sync_copy(ui.at[pl.ds(base, bpw)], uidx_v)
        pltpu.sync_copy(ii.at[pl.ds(base, bpw)], iidx_v)
        gathers = []
        for t, (idx_v, rows_v) in enumerate(((uidx_v, urows_v), (iidx_v, irows_v))):
            tbl = ut if t == 0 else it
            for j in range(nchunk):
                gathers.append((t, j, pltpu.async_copy(
                    tbl.at[idx_v.at[pl.ds(j * _CHUNK, _CHUNK)]],
                    rows_v.at[pl.ds(j * _CHUNK, _CHUNK)], gsem.at[t, j])))
        writes = []
        for t, j, g in gathers:
            g.wait()
            rows_v = urows_v if t == 0 else irows_v
            writes.append(pltpu.async_copy(
                rows_v.at[pl.ds(j * _CHUNK, _CHUNK)],
                out.at[t, pl.ds(base + j * _CHUNK, _CHUNK)], wsem))
        for w in writes:
            w.wait()

    return _emb(user_table, item_table,
                user_idx.astype(jnp.int32), item_idx.astype(jnp.int32))


# speedup vs baseline: 1.4102x; 1.4102x over previous
import functools

import jax
import jax.numpy as jnp
from jax import lax
from jax.experimental import pallas as pl
from jax.experimental.pallas import tpu as pltpu
from jax.experimental.pallas import tpu_sc as plsc

_B = 16384
_D = 64
_CHUNK = 256


def kernel(user_table, item_table, user_idx, item_idx):
    info = plsc.get_sparse_core_info()
    nw = info.num_cores * info.num_subcores
    bpw = _B // nw

    mesh = plsc.VectorSubcoreMesh(core_axis_name="c", subcore_axis_name="s")

    @functools.partial(
        pl.kernel,
        mesh=mesh,
        out_type=jax.ShapeDtypeStruct((2, _B, _D), jnp.float32),
        scratch_types=[
            pltpu.VMEM((bpw,), jnp.int32),
            pltpu.VMEM((bpw,), jnp.int32),
            pltpu.VMEM((_CHUNK, _D), jnp.float32),
            pltpu.VMEM((_CHUNK, _D), jnp.float32),
            pltpu.SemaphoreType.DMA,
            pltpu.SemaphoreType.DMA,
        ],
        compiler_params=pltpu.CompilerParams(use_tc_tiling_on_sc=True),
    )
    def _emb(ut, it, ui, ii, out, uidx_s, iidx_s, urows_v, irows_v,
             usem, isem):
        wid = lax.axis_index("s") * info.num_cores + lax.axis_index("c")
        base = wid * bpw
        pltpu.sync_copy(ui.at[pl.ds(base, bpw)], uidx_s)
        pltpu.sync_copy(ii.at[pl.ds(base, bpw)], iidx_s)

        def enqueue(tbl, idx_s, rows_v, sem, c):
            def body(g, carry):
                vec = idx_s[pl.ds(c * _CHUNK + g * 16, 16)]
                for k in range(16):
                    pltpu.async_copy(tbl.at[vec[k]], rows_v.at[g * 16 + k],
                                     sem)
                return carry
            lax.fori_loop(0, _CHUNK // 16, body, 0)

        nchunk = bpw // _CHUNK
        enqueue(ut, uidx_s, urows_v, usem, 0)
        enqueue(it, iidx_s, irows_v, isem, 0)
        for c in range(nchunk):
            pltpu.make_async_copy(ut.at[pl.ds(0, _CHUNK)], urows_v, usem).wait()
            pltpu.sync_copy(urows_v,
                            out.at[0, pl.ds(base + c * _CHUNK, _CHUNK)])
            if c + 1 < nchunk:
                enqueue(ut, uidx_s, urows_v, usem, c + 1)
            pltpu.make_async_copy(it.at[pl.ds(0, _CHUNK)], irows_v, isem).wait()
            pltpu.sync_copy(irows_v,
                            out.at[1, pl.ds(base + c * _CHUNK, _CHUNK)])
            if c + 1 < nchunk:
                enqueue(it, iidx_s, irows_v, isem, c + 1)

    return _emb(user_table, item_table,
                user_idx.astype(jnp.int32), item_idx.astype(jnp.int32))
